# Initial kernel scaffold; baseline (speedup 1.0000x reference)
import functools

import jax
import jax.numpy as jnp
from jax import lax
from jax.experimental import pallas as pl
from jax.experimental.pallas import tpu as pltpu
from jax.experimental.pallas import tpu_sc as plsc

N_NODES = 50000
N_EDGES = 800000
CHANNELS = 64
HALF_C = 32
NC = 2
NS = 16
LANES = 16

CHUNK = 128
E_PER_SUB = 50048
N_CHUNKS = E_PER_SUB // CHUNK
E_PAD = E_PER_SUB * NS

HALF_N = N_NODES // NC
ACC_ROWS = 50016
CNT_ROWS = 25008
ACC_DUMMY = N_NODES
CNT_DUMMY = HALF_N
PR_ACC = ACC_ROWS // NS
PR_CNT = CNT_ROWS // NS


def _sc_body(xcat, srcs, tgts, zacc, zcnt, ones_hbm,
             sums_out, cnts_out,
             acc_sp, cnt_sp, rows_v, ones_v, src_v, tgt_v, asrc_v, cidx_v):
  c = lax.axis_index("c")
  s = lax.axis_index("s")

  pltpu.sync_copy(zacc.at[pl.ds(s * PR_ACC, PR_ACC)],
                  acc_sp.at[pl.ds(s * PR_ACC, PR_ACC)])
  pltpu.sync_copy(zcnt.at[pl.ds(s * PR_CNT, PR_CNT)],
                  cnt_sp.at[pl.ds(s * PR_CNT, PR_CNT)])
  pltpu.sync_copy(ones_hbm, ones_v)
  plsc.subcore_barrier()

  base = s * E_PER_SUB
  c_src_off = c * N_NODES
  c_cnt_off = c * HALF_N

  @pl.loop(0, N_CHUNKS)
  def _(i):
    off = base + i * CHUNK
    pltpu.sync_copy(srcs.at[pl.ds(off, CHUNK)], src_v)
    pltpu.sync_copy(tgts.at[pl.ds(off, CHUNK)], tgt_v)

    @pl.loop(0, CHUNK // LANES)
    def _(j):
      sl = pl.ds(j * LANES, LANES)
      asrc_v[sl] = src_v[sl] + c_src_off
      u = tgt_v[sl] - c_cnt_off
      cidx_v[sl] = jnp.where(u < 0, CNT_DUMMY, jnp.minimum(u, CNT_DUMMY))

    pltpu.sync_copy(xcat.at[asrc_v], rows_v)
    pltpu.sync_copy(rows_v, acc_sp.at[tgt_v], add=True)
    pltpu.sync_copy(ones_v, cnt_sp.at[cidx_v], add=True)

  plsc.subcore_barrier()

  pltpu.sync_copy(acc_sp.at[pl.ds(s * PR_ACC, PR_ACC)],
                  sums_out.at[pl.ds(c * ACC_ROWS + s * PR_ACC, PR_ACC)])
  pltpu.sync_copy(cnt_sp.at[pl.ds(s * PR_CNT, PR_CNT)],
                  cnts_out.at[pl.ds(c * CNT_ROWS + s * PR_CNT, PR_CNT)])


def _sc_accumulate(xcat, srcs, tgts, zacc, zcnt, ones):
  mesh = plsc.VectorSubcoreMesh(core_axis_name="c", subcore_axis_name="s",
                                num_cores=NC, num_subcores=NS)
  return pl.kernel(
      _sc_body,
      out_type=[
          jax.ShapeDtypeStruct((NC * ACC_ROWS, HALF_C), jnp.float32),
          jax.ShapeDtypeStruct((NC * CNT_ROWS, LANES), jnp.float32),
      ],
      mesh=mesh,
      scratch_types=[
          pltpu.VMEM_SHARED((ACC_ROWS, HALF_C), jnp.float32),
          pltpu.VMEM_SHARED((CNT_ROWS, LANES), jnp.float32),
          pltpu.VMEM((CHUNK, HALF_C), jnp.float32),
          pltpu.VMEM((CHUNK, LANES), jnp.float32),
          pltpu.VMEM((CHUNK,), jnp.int32),
          pltpu.VMEM((CHUNK,), jnp.int32),
          pltpu.VMEM((CHUNK,), jnp.int32),
          pltpu.VMEM((CHUNK,), jnp.int32),
      ],
  )(xcat, srcs, tgts, zacc, zcnt, ones)


BR = 2500
NB = N_NODES // BR
_BLOCKS_PER_HALF = HALF_N // BR


def _tc_mean_stats_body(sums_ref, cnt_ref, m_ref, stats_ref):
  i = pl.program_id(0)

  cnt = jnp.clip(cnt_ref[0, :, 0:1], 1.0, None)
  m = jnp.concatenate([sums_ref[0], sums_ref[1]], axis=1) / cnt
  m_ref[...] = m

  @pl.when(i == 0)
  def _():
    stats_ref[...] = jnp.zeros_like(stats_ref)

  stats_ref[0:1, :] += jnp.sum(m, axis=0, keepdims=True)
  stats_ref[1:2, :] += jnp.sum(m * m, axis=0, keepdims=True)


def _tc_mean_stats(sums, cnts):
  sums3 = sums.reshape(NC, ACC_ROWS, HALF_C)
  cnts3 = cnts.reshape(NC, CNT_ROWS, LANES)
  return pl.pallas_call(
      _tc_mean_stats_body,
      grid=(NB,),
      in_specs=[
          pl.BlockSpec((NC, BR, HALF_C), lambda i: (0, i, 0)),
          pl.BlockSpec((1, BR, LANES),
                       lambda i: (i // _BLOCKS_PER_HALF, i % _BLOCKS_PER_HALF, 0)),
      ],
      out_specs=[
          pl.BlockSpec((BR, CHANNELS), lambda i: (i, 0)),
          pl.BlockSpec((8, CHANNELS), lambda i: (0, 0)),
      ],
      out_shape=[
          jax.ShapeDtypeStruct((N_NODES, CHANNELS), jnp.float32),
          jax.ShapeDtypeStruct((8, CHANNELS), jnp.float32),
      ],
  )(sums3, cnts3)


def _tc_bn_linear_body(m_ref, stats_ref, bnw_ref, bnb_ref, wt_ref, b_ref, o_ref):
  inv_n = 1.0 / N_NODES
  mu = stats_ref[0:1, :] * inv_n
  var = stats_ref[1:2, :] * inv_n - mu * mu
  scale = bnw_ref[0:1, :] * lax.rsqrt(var + 1e-5)
  shift = bnb_ref[0:1, :] - mu * scale
  h = m_ref[...] * scale + shift
  out = lax.dot_general(h, wt_ref[...], (((1,), (0,)), ((), ())),
                        precision=lax.Precision.HIGHEST,
                        preferred_element_type=jnp.float32)
  o_ref[...] = jnp.maximum(out + b_ref[0:1, :], 0.0)


def _tc_bn_linear(m, stats, bn_weight, bn_bias, w_t, b):
  return pl.pallas_call(
      _tc_bn_linear_body,
      grid=(NB,),
      in_specs=[
          pl.BlockSpec((BR, CHANNELS), lambda i: (i, 0)),
          pl.BlockSpec((8, CHANNELS), lambda i: (0, 0)),
          pl.BlockSpec((1, CHANNELS), lambda i: (0, 0)),
          pl.BlockSpec((1, CHANNELS), lambda i: (0, 0)),
          pl.BlockSpec((CHANNELS, CHANNELS), lambda i: (0, 0)),
          pl.BlockSpec((1, CHANNELS), lambda i: (0, 0)),
      ],
      out_specs=pl.BlockSpec((BR, CHANNELS), lambda i: (i, 0)),
      out_shape=jax.ShapeDtypeStruct((N_NODES, CHANNELS), jnp.float32),
  )(m, stats, bn_weight.reshape(1, CHANNELS), bn_bias.reshape(1, CHANNELS),
    w_t, b.reshape(1, CHANNELS))


@jax.jit
def kernel(x, sources, targets, bn_weight, bn_bias, W, b):
  s32 = sources.astype(jnp.int32)
  t32 = targets.astype(jnp.int32)
  pad = E_PAD - N_EDGES
  srcs = jnp.concatenate([s32, jnp.zeros((pad,), jnp.int32)])
  tgts = jnp.concatenate([t32, jnp.full((pad,), ACC_DUMMY, jnp.int32)])
  xcat = jnp.concatenate([x[:, :HALF_C], x[:, HALF_C:]], axis=0)
  zacc = jnp.zeros((ACC_ROWS, HALF_C), jnp.float32)
  zcnt = jnp.zeros((CNT_ROWS, LANES), jnp.float32)
  ones = jnp.ones((CHUNK, LANES), jnp.float32)

  sums, cnts = _sc_accumulate(xcat, srcs, tgts, zacc, zcnt, ones)
  m, stats = _tc_mean_stats(sums, cnts)
  return _tc_bn_linear(m, stats, bn_weight, bn_bias, W.T, b)


# speedup vs baseline: 4.6566x; 4.6566x over previous
import jax
import jax.numpy as jnp
from jax import lax
from jax.experimental import pallas as pl
from jax.experimental.pallas import tpu as pltpu
from jax.experimental.pallas import tpu_sc as plsc

N_NODES = 50000
N_EDGES = 800000
CHANNELS = 64
HALF_C = 32
NC = 2
NS = 16
LANES = 16

CHUNK = 128
CNT_SUB = 32
E_PER_SUB = 50048
N_CHUNKS = E_PER_SUB // CHUNK
E_PAD = E_PER_SUB * NS

HALF_N = N_NODES // NC
ACC_ROWS = 50016
CNT_ROWS = 25008
ACC_DUMMY = N_NODES
CNT_DUMMY = HALF_N
PR_ACC = ACC_ROWS // NS
PR_CNT = CNT_ROWS // NS


def _sc_body(xcat, srcs, tgts, zacc, zcnt, ones_hbm,
             sums_out, cnts_out,
             acc_sp, cnt_sp, rows_v, ones_v, src_v, tgt_v, cidx_v):
  c = lax.axis_index("c")
  s = lax.axis_index("s")

  pltpu.sync_copy(zacc.at[pl.ds(s * PR_ACC, PR_ACC)],
                  acc_sp.at[pl.ds(s * PR_ACC, PR_ACC)])
  pltpu.sync_copy(zcnt.at[pl.ds(s * PR_CNT, PR_CNT)],
                  cnt_sp.at[pl.ds(s * PR_CNT, PR_CNT)])
  pltpu.sync_copy(ones_hbm, ones_v)
  plsc.subcore_barrier()

  base = s * E_PER_SUB
  c_src_off = c * N_NODES
  c_cnt_off = c * HALF_N

  @pl.loop(0, N_CHUNKS)
  def _(i):
    off = base + i * CHUNK
    pltpu.sync_copy(srcs.at[pl.ds(off, CHUNK)], src_v)
    pltpu.sync_copy(tgts.at[pl.ds(off, CHUNK)], tgt_v)

    @pl.loop(0, CHUNK // LANES)
    def _(j):
      sl = pl.ds(j * LANES, LANES)
      src_v[sl] = src_v[sl] + c_src_off
      u = tgt_v[sl] - c_cnt_off
      cidx_v[j // 2, pl.ds((j % 2) * LANES, LANES)] = (
          jnp.where(u < 0, CNT_DUMMY, jnp.minimum(u, CNT_DUMMY)))

    pltpu.sync_copy(xcat.at[src_v], rows_v)
    pltpu.sync_copy(rows_v, acc_sp.at[tgt_v], add=True)
    @pl.loop(0, CHUNK // CNT_SUB)
    def _(j):
      pltpu.sync_copy(ones_v, cnt_sp.at[cidx_v.at[j]], add=True)

  plsc.subcore_barrier()

  pltpu.sync_copy(acc_sp.at[pl.ds(s * PR_ACC, PR_ACC)],
                  sums_out.at[pl.ds(c * ACC_ROWS + s * PR_ACC, PR_ACC)])
  pltpu.sync_copy(cnt_sp.at[pl.ds(s * PR_CNT, PR_CNT)],
                  cnts_out.at[pl.ds(c * CNT_ROWS + s * PR_CNT, PR_CNT)])


def _sc_accumulate(xcat, srcs, tgts, zacc, zcnt, ones):
  mesh = plsc.VectorSubcoreMesh(core_axis_name="c", subcore_axis_name="s",
                                num_cores=NC, num_subcores=NS)
  return pl.kernel(
      _sc_body,
      compiler_params=pltpu.CompilerParams(use_tc_tiling_on_sc=False),
      out_type=[
          jax.ShapeDtypeStruct((NC * ACC_ROWS, HALF_C), jnp.float32),
          jax.ShapeDtypeStruct((NC * CNT_ROWS, LANES), jnp.float32),
      ],
      mesh=mesh,
      scratch_types=[
          pltpu.VMEM_SHARED((ACC_ROWS, HALF_C), jnp.float32),
          pltpu.VMEM_SHARED((CNT_ROWS, LANES), jnp.float32),
          pltpu.VMEM((CHUNK, HALF_C), jnp.float32),
          pltpu.VMEM((CNT_SUB, LANES), jnp.float32),
          pltpu.VMEM((CHUNK,), jnp.int32),
          pltpu.VMEM((CHUNK,), jnp.int32),
          pltpu.VMEM((CHUNK // CNT_SUB, CNT_SUB), jnp.int32),
      ],
  )(xcat, srcs, tgts, zacc, zcnt, ones)


BR = 5000
NB = N_NODES // BR
_BLOCKS_PER_HALF = HALF_N // BR


def _tc_mean_stats_body(sums_ref, cnt_ref, m_ref, stats_ref):
  i = pl.program_id(0)

  cnt = jnp.clip(cnt_ref[0, :, 0:1], 1.0, None)
  m = jnp.concatenate([sums_ref[0], sums_ref[1]], axis=1) / cnt
  m_ref[...] = m

  @pl.when(i == 0)
  def _():
    stats_ref[...] = jnp.zeros_like(stats_ref)

  stats_ref[0:1, :] += jnp.sum(m, axis=0, keepdims=True)
  stats_ref[1:2, :] += jnp.sum(m * m, axis=0, keepdims=True)


def _tc_mean_stats(sums, cnts):
  sums3 = sums.reshape(NC, ACC_ROWS, HALF_C)
  cnts3 = cnts.reshape(NC, CNT_ROWS, LANES)
  return pl.pallas_call(
      _tc_mean_stats_body,
      grid=(NB,),
      in_specs=[
          pl.BlockSpec((NC, BR, HALF_C), lambda i: (0, i, 0)),
          pl.BlockSpec((1, BR, LANES),
                       lambda i: (i // _BLOCKS_PER_HALF, i % _BLOCKS_PER_HALF, 0)),
      ],
      out_specs=[
          pl.BlockSpec((BR, CHANNELS), lambda i: (i, 0)),
          pl.BlockSpec((8, CHANNELS), lambda i: (0, 0)),
      ],
      out_shape=[
          jax.ShapeDtypeStruct((N_NODES, CHANNELS), jnp.float32),
          jax.ShapeDtypeStruct((8, CHANNELS), jnp.float32),
      ],
  )(sums3, cnts3)


def _tc_bn_linear_body(m_ref, stats_ref, bnw_ref, bnb_ref, wt_ref, b_ref, o_ref):
  inv_n = 1.0 / N_NODES
  mu = stats_ref[0:1, :] * inv_n
  var = stats_ref[1:2, :] * inv_n - mu * mu
  scale = bnw_ref[0:1, :] * lax.rsqrt(var + 1e-5)
  shift = bnb_ref[0:1, :] - mu * scale
  h = m_ref[...] * scale + shift
  out = lax.dot_general(h, wt_ref[...], (((1,), (0,)), ((), ())),
                        precision=lax.Precision.HIGHEST,
                        preferred_element_type=jnp.float32)
  o_ref[...] = jnp.maximum(out + b_ref[0:1, :], 0.0)


def _tc_bn_linear(m, stats, bn_weight, bn_bias, w_t, b):
  return pl.pallas_call(
      _tc_bn_linear_body,
      grid=(NB,),
      in_specs=[
          pl.BlockSpec((BR, CHANNELS), lambda i: (i, 0)),
          pl.BlockSpec((8, CHANNELS), lambda i: (0, 0)),
          pl.BlockSpec((1, CHANNELS), lambda i: (0, 0)),
          pl.BlockSpec((1, CHANNELS), lambda i: (0, 0)),
          pl.BlockSpec((CHANNELS, CHANNELS), lambda i: (0, 0)),
          pl.BlockSpec((1, CHANNELS), lambda i: (0, 0)),
      ],
      out_specs=pl.BlockSpec((BR, CHANNELS), lambda i: (i, 0)),
      out_shape=jax.ShapeDtypeStruct((N_NODES, CHANNELS), jnp.float32),
  )(m, stats, bn_weight.reshape(1, CHANNELS), bn_bias.reshape(1, CHANNELS),
    w_t, b.reshape(1, CHANNELS))


@jax.jit
def kernel(x, sources, targets, bn_weight, bn_bias, W, b):
  s32 = sources.astype(jnp.int32)
  t32 = targets.astype(jnp.int32)
  pad = E_PAD - N_EDGES
  srcs = jnp.concatenate([s32, jnp.zeros((pad,), jnp.int32)])
  tgts = jnp.concatenate([t32, jnp.full((pad,), ACC_DUMMY, jnp.int32)])
  xcat = jnp.concatenate([x[:, :HALF_C], x[:, HALF_C:]], axis=0)
  zacc = jnp.zeros((ACC_ROWS, HALF_C), jnp.float32)
  zcnt = jnp.zeros((CNT_ROWS, LANES), jnp.float32)
  ones = jnp.ones((CNT_SUB, LANES), jnp.float32)

  sums, cnts = _sc_accumulate(xcat, srcs, tgts, zacc, zcnt, ones)
  m, stats = _tc_mean_stats(sums, cnts)
  return _tc_bn_linear(m, stats, bn_weight, bn_bias, W.T, b)


# speedup vs baseline: 5.9288x; 1.2732x over previous
import jax
import jax.numpy as jnp
from jax import lax
from jax.experimental import pallas as pl
from jax.experimental.pallas import tpu as pltpu
from jax.experimental.pallas import tpu_sc as plsc

N_NODES = 50000
N_EDGES = 800000
CHANNELS = 64
HALF_C = 32
NC = 2
NS = 16
LANES = 16

CHUNK = 128
E_PER_SUB = 50048
N_CHUNKS = E_PER_SUB // CHUNK
E_PAD = E_PER_SUB * NS

HALF_N = N_NODES // NC
ACC_ROWS = 50016
CNT_ROWS = 25008
ACC_DUMMY = N_NODES
CNT_DUMMY = HALF_N
PR_ACC = ACC_ROWS // NS
PR_CNT = CNT_ROWS // NS

NBUF_R = 4
NBUF_I = 6
STEP = 12
LOOP_HI = 408


def _sc_body(xcat, srcs, tgts, zacc, ones_hbm, sums_out, cnts_out, *scratch):
  acc_sp = scratch[0]
  rows = scratch[1:1 + NBUF_R]
  src = scratch[5:5 + NBUF_I]
  tgt = scratch[11:11 + NBUF_I]
  si = scratch[17:17 + NBUF_I]
  sg = scratch[23:23 + NBUF_R]
  ss = scratch[27:27 + NBUF_R]

  c = lax.axis_index("c")
  s = lax.axis_index("s")

  pltpu.sync_copy(zacc.at[pl.ds(s * PR_ACC, PR_ACC)],
                  acc_sp.at[pl.ds(s * PR_ACC, PR_ACC)])
  plsc.subcore_barrier()

  base = s * E_PER_SUB
  c_src_off = c * N_NODES
  c_cnt_off = c * HALF_N

  def issue_idx(m, j, with_src):
    off = base + m * CHUNK
    if with_src:
      pltpu.async_copy(srcs.at[pl.ds(off, CHUNK)], src[j], si[j])
    pltpu.async_copy(tgts.at[pl.ds(off, CHUNK)], tgt[j], si[j])

  def wait_idx(j, with_src):
    if with_src:
      pltpu.make_async_copy(srcs.at[pl.ds(0, CHUNK)], src[j], si[j]).wait()
    pltpu.make_async_copy(tgts.at[pl.ds(0, CHUNK)], tgt[j], si[j]).wait()

  def wait_rows(k, sem):
    pltpu.make_async_copy(ones_hbm, rows[k], sem).wait()

  issue_idx(0, 0, True)
  issue_idx(1, 1, True)

  @pl.loop(0, LOOP_HI, step=STEP)
  def _(i):
    for b in range(STEP):
      m = i + b
      jb = b % NBUF_I
      kb = b % NBUF_R
      k2 = (b - 2) % NBUF_R
      k4 = (b - 4) % NBUF_R
      j2 = (b - 2) % NBUF_I
      jp2 = (b + 2) % NBUF_I

      @pl.when(jnp.logical_and(m >= 4, m < N_CHUNKS + 4))
      def _():
        wait_rows(k4, ss[k4])

      @pl.when(jnp.logical_and(m >= 2, m < N_CHUNKS + 2))
      def _():
        wait_rows(k2, sg[k2])
        pltpu.async_copy(rows[k2], acc_sp.at[tgt[j2]], ss[k2], add=True)

      @pl.when(m + 2 < N_CHUNKS)
      def _():
        issue_idx(m + 2, jp2, True)

      @pl.when(m < N_CHUNKS)
      def _():
        wait_idx(jb, True)

        @pl.loop(0, CHUNK // LANES)
        def _(q):
          sl = pl.ds(q * LANES, LANES)
          src[jb][sl] = src[jb][sl] + c_src_off

        pltpu.async_copy(xcat.at[src[jb]], rows[kb], sg[kb])

  plsc.subcore_barrier()

  pltpu.sync_copy(acc_sp.at[pl.ds(s * PR_ACC, PR_ACC)],
                  sums_out.at[pl.ds(c * ACC_ROWS + s * PR_ACC, PR_ACC)])
  plsc.subcore_barrier()

  pltpu.sync_copy(zacc.at[pl.ds(s * PR_CNT, PR_CNT)],
                  acc_sp.at[pl.ds(s * PR_CNT, PR_CNT)])
  pltpu.sync_copy(ones_hbm, rows[0])
  plsc.subcore_barrier()

  issue_idx(0, 0, False)
  issue_idx(1, 1, False)

  @pl.loop(0, LOOP_HI, step=STEP)
  def _(i):
    for b in range(STEP):
      m = i + b
      jb = b % NBUF_I
      kb = b % NBUF_R
      k4 = (b - 4) % NBUF_R
      jp2 = (b + 2) % NBUF_I

      @pl.when(jnp.logical_and(m >= 4, m < N_CHUNKS + 4))
      def _():
        wait_rows(1, ss[k4])

      @pl.when(m + 2 < N_CHUNKS)
      def _():
        issue_idx(m + 2, jp2, False)

      @pl.when(m < N_CHUNKS)
      def _():
        wait_idx(jb, False)

        @pl.loop(0, CHUNK // LANES)
        def _(q):
          sl = pl.ds(q * LANES, LANES)
          u = tgt[jb][sl] - c_cnt_off
          src[kb][sl] = jnp.where(u < 0, CNT_DUMMY, jnp.minimum(u, CNT_DUMMY))

        pltpu.async_copy(rows[0], acc_sp.at[src[kb]], ss[kb], add=True)

  plsc.subcore_barrier()

  pltpu.sync_copy(acc_sp.at[pl.ds(s * PR_CNT, PR_CNT)],
                  cnts_out.at[pl.ds(c * CNT_ROWS + s * PR_CNT, PR_CNT)])


def _sc_accumulate(xcat, srcs, tgts, zacc, ones):
  mesh = plsc.VectorSubcoreMesh(core_axis_name="c", subcore_axis_name="s",
                                num_cores=NC, num_subcores=NS)
  scratch = (
      [pltpu.VMEM_SHARED((ACC_ROWS, HALF_C), jnp.float32)]
      + [pltpu.VMEM((CHUNK, HALF_C), jnp.float32) for _ in range(NBUF_R)]
      + [pltpu.VMEM((CHUNK,), jnp.int32) for _ in range(2 * NBUF_I)]
      + [pltpu.SemaphoreType.DMA for _ in range(NBUF_I + 2 * NBUF_R)]
  )
  return pl.kernel(
      _sc_body,
      compiler_params=pltpu.CompilerParams(use_tc_tiling_on_sc=False),
      out_type=[
          jax.ShapeDtypeStruct((NC * ACC_ROWS, HALF_C), jnp.float32),
          jax.ShapeDtypeStruct((NC * CNT_ROWS, HALF_C), jnp.float32),
      ],
      mesh=mesh,
      scratch_types=scratch,
  )(xcat, srcs, tgts, zacc, ones)


BR = 5000
NB = N_NODES // BR
_BLOCKS_PER_HALF = HALF_N // BR


def _tc_mean_stats_body(sums_ref, cnt_ref, m_ref, stats_ref):
  i = pl.program_id(0)

  cnt = jnp.clip(cnt_ref[0, :, 0:1], 1.0, None)
  m = jnp.concatenate([sums_ref[0], sums_ref[1]], axis=1) / cnt
  m_ref[...] = m

  @pl.when(i == 0)
  def _():
    stats_ref[...] = jnp.zeros_like(stats_ref)

  stats_ref[0:1, :] += jnp.sum(m, axis=0, keepdims=True)
  stats_ref[1:2, :] += jnp.sum(m * m, axis=0, keepdims=True)


def _tc_mean_stats(sums, cnts):
  sums3 = sums.reshape(NC, ACC_ROWS, HALF_C)
  cnts3 = cnts.reshape(NC, CNT_ROWS, HALF_C)
  return pl.pallas_call(
      _tc_mean_stats_body,
      grid=(NB,),
      in_specs=[
          pl.BlockSpec((NC, BR, HALF_C), lambda i: (0, i, 0)),
          pl.BlockSpec((1, BR, HALF_C),
                       lambda i: (i // _BLOCKS_PER_HALF, i % _BLOCKS_PER_HALF, 0)),
      ],
      out_specs=[
          pl.BlockSpec((BR, CHANNELS), lambda i: (i, 0)),
          pl.BlockSpec((8, CHANNELS), lambda i: (0, 0)),
      ],
      out_shape=[
          jax.ShapeDtypeStruct((N_NODES, CHANNELS), jnp.float32),
          jax.ShapeDtypeStruct((8, CHANNELS), jnp.float32),
      ],
  )(sums3, cnts3)


def _tc_bn_linear_body(m_ref, stats_ref, bnw_ref, bnb_ref, wt_ref, b_ref, o_ref):
  inv_n = 1.0 / N_NODES
  mu = stats_ref[0:1, :] * inv_n
  var = stats_ref[1:2, :] * inv_n - mu * mu
  scale = bnw_ref[0:1, :] * lax.rsqrt(var + 1e-5)
  shift = bnb_ref[0:1, :] - mu * scale
  h = m_ref[...] * scale + shift
  out = lax.dot_general(h, wt_ref[...], (((1,), (0,)), ((), ())),
                        precision=lax.Precision.HIGHEST,
                        preferred_element_type=jnp.float32)
  o_ref[...] = jnp.maximum(out + b_ref[0:1, :], 0.0)


def _tc_bn_linear(m, stats, bn_weight, bn_bias, w_t, b):
  return pl.pallas_call(
      _tc_bn_linear_body,
      grid=(NB,),
      in_specs=[
          pl.BlockSpec((BR, CHANNELS), lambda i: (i, 0)),
          pl.BlockSpec((8, CHANNELS), lambda i: (0, 0)),
          pl.BlockSpec((1, CHANNELS), lambda i: (0, 0)),
          pl.BlockSpec((1, CHANNELS), lambda i: (0, 0)),
          pl.BlockSpec((CHANNELS, CHANNELS), lambda i: (0, 0)),
          pl.BlockSpec((1, CHANNELS), lambda i: (0, 0)),
      ],
      out_specs=pl.BlockSpec((BR, CHANNELS), lambda i: (i, 0)),
      out_shape=jax.ShapeDtypeStruct((N_NODES, CHANNELS), jnp.float32),
  )(m, stats, bn_weight.reshape(1, CHANNELS), bn_bias.reshape(1, CHANNELS),
    w_t, b.reshape(1, CHANNELS))


@jax.jit
def kernel(x, sources, targets, bn_weight, bn_bias, W, b):
  s32 = sources.astype(jnp.int32)
  t32 = targets.astype(jnp.int32)
  pad = E_PAD - N_EDGES
  srcs = jnp.concatenate([s32, jnp.zeros((pad,), jnp.int32)])
  tgts = jnp.concatenate([t32, jnp.full((pad,), ACC_DUMMY, jnp.int32)])
  xcat = jnp.concatenate([x[:, :HALF_C], x[:, HALF_C:]], axis=0)
  zacc = jnp.zeros((ACC_ROWS, HALF_C), jnp.float32)
  ones = jnp.ones((CHUNK, HALF_C), jnp.float32)

  sums, cnts = _sc_accumulate(xcat, srcs, tgts, zacc, ones)
  m, stats = _tc_mean_stats(sums, cnts)
  return _tc_bn_linear(m, stats, bn_weight, bn_bias, W.T, b)
